# baseline (speedup 1.0000x reference)
import functools

import jax
import jax.numpy as jnp
from jax import lax
from jax.experimental import pallas as pl
from jax.experimental.pallas import tpu as pltpu
from jax.experimental.pallas import tpu_sc as plsc

_NUM_CORES = 2
_NUM_SUBCORES = 16
_NW = _NUM_CORES * _NUM_SUBCORES
_LANES = 16


def _make_pool(ctx, b, d):
    rows_per_w = b // _NW
    n_cvec = d // _LANES
    mesh = plsc.VectorSubcoreMesh(
        core_axis_name="c", subcore_axis_name="s"
    )

    @functools.partial(
        pl.kernel,
        out_type=jax.ShapeDtypeStruct((b, d), jnp.float32),
        mesh=mesh,
        scratch_types=[
            pltpu.VMEM((rows_per_w,), jnp.int32),
            pltpu.VMEM((rows_per_w, d), jnp.float32),
            pltpu.VMEM((rows_per_w, d), jnp.float32),
            pltpu.SemaphoreType.DMA,
        ],
        compiler_params=pltpu.CompilerParams(use_tc_tiling_on_sc=False),
    )
    def pool(xT_hbm, table_hbm, h_hbm, idx_v, rows_v, acc_v, sem):
        wid = lax.axis_index("s") * _NUM_CORES + lax.axis_index("c")
        base = wid * rows_per_w

        pltpu.sync_copy(xT_hbm.at[0, pl.ds(base, rows_per_w)], idx_v)
        pltpu.async_copy(table_hbm.at[idx_v], acc_v, sem).wait()

        for j in range(1, ctx):
            pltpu.sync_copy(xT_hbm.at[j, pl.ds(base, rows_per_w)], idx_v)
            pltpu.async_copy(table_hbm.at[idx_v], rows_v, sem).wait()

            def add_row(r, carry):
                for c in range(n_cvec):
                    sl = pl.ds(c * _LANES, _LANES)
                    acc_v[r, sl] += rows_v[r, sl]
                return carry

            lax.fori_loop(0, rows_per_w, add_row, 0)

        pltpu.sync_copy(acc_v, h_hbm.at[pl.ds(base, rows_per_w)])

    return pool


def _matmul_body(scale, h_ref, w_ref, b_ref, out_ref):
    h = h_ref[...] * scale
    out_ref[...] = (
        lax.dot_general(
            h,
            w_ref[...],
            dimension_numbers=(((1,), (1,)), ((), ())),
            preferred_element_type=jnp.float32,
        )
        + b_ref[...]
    )


def _projection(h_sum, w, b2d, ctx, bb, vb):
    batch, d = h_sum.shape
    vocab = w.shape[0]
    nb = batch // bb
    nv = pl.cdiv(vocab, vb)
    return pl.pallas_call(
        functools.partial(_matmul_body, float(1.0 / ctx)),
        grid=(nv, nb),
        in_specs=[
            pl.BlockSpec((bb, d), lambda j, i: (i, 0)),
            pl.BlockSpec((vb, d), lambda j, i: (j, 0)),
            pl.BlockSpec((1, vb), lambda j, i: (0, j)),
        ],
        out_specs=pl.BlockSpec((bb, vb), lambda j, i: (i, j)),
        out_shape=jax.ShapeDtypeStruct((batch, vocab), jnp.float32),
        compiler_params=pltpu.CompilerParams(
            dimension_semantics=("arbitrary", "arbitrary"),
        ),
    )(h_sum, w, b2d)


def kernel(x, emb_table, W, b):
    batch, ctx = x.shape
    vocab, d = W.shape
    xT = x.T
    h_sum = _make_pool(ctx, batch, d)(xT, emb_table)
    return _projection(h_sum, W, b.reshape(1, vocab), ctx, 512, 2048)
